# baseline (speedup 1.0000x reference)
import numpy as np
import jax
import jax.numpy as jnp
from jax.experimental import pallas as pl


_S = 4


def _shifts():
    rng = np.random.default_rng(0)
    h_shift = float(rng.random() * 2 * _S - _S)
    w_shift = float(rng.random() * 2 * _S - _S)
    return h_shift, w_shift


def _weights(n, shift):
    idx = jnp.arange(n, dtype=jnp.int32)
    shifted = jnp.clip(idx.astype(jnp.float32) + shift, 0.0, float(n - 1))
    f = jnp.floor(shifted)
    c = jnp.ceil(shifted)
    w_c = c - shifted
    w_f = shifted - f
    return w_c, w_f


def _stencil_kernel(x_ref, whc_ref, whf_ref, wwc_ref, wwf_ref, o_ref):
    x = x_ref[...]
    t = wwc_ref[...] * jnp.roll(x, 2, axis=2) + wwf_ref[...] * jnp.roll(x, 1, axis=2)
    o_ref[...] = whc_ref[...] * jnp.roll(t, -1, axis=1) + whf_ref[...] * jnp.roll(t, -2, axis=1)


def kernel(x):
    h_shift, w_shift = _shifts()
    B, C, H, W = x.shape
    wh_c, wh_f = _weights(H, h_shift)
    ww_c, ww_f = _weights(W, w_shift)

    n = B * C
    xr = x.reshape(n, H, W)
    blk = 16
    grid = (n // blk,)

    out = pl.pallas_call(
        _stencil_kernel,
        grid=grid,
        in_specs=[
            pl.BlockSpec((blk, H, W), lambda i: (i, 0, 0)),
            pl.BlockSpec((1, H, 1), lambda i: (0, 0, 0)),
            pl.BlockSpec((1, H, 1), lambda i: (0, 0, 0)),
            pl.BlockSpec((1, 1, W), lambda i: (0, 0, 0)),
            pl.BlockSpec((1, 1, W), lambda i: (0, 0, 0)),
        ],
        out_specs=pl.BlockSpec((blk, H, W), lambda i: (i, 0, 0)),
        out_shape=jax.ShapeDtypeStruct((n, H, W), x.dtype),
    )(
        xr,
        wh_c.reshape(1, H, 1),
        wh_f.reshape(1, H, 1),
        ww_c.reshape(1, 1, W),
        ww_f.reshape(1, 1, W),
    )
    return out.reshape(B, C, H, W)
